# baseline (speedup 1.0000x reference)
import functools

import jax
import jax.numpy as jnp
from jax import lax
from jax.experimental import pallas as pl
from jax.experimental.pallas import tpu as pltpu
from jax.experimental.pallas import tpu_sc as plsc

B = 16384
D = 128
GAMMA = 12.0

NC = 2
NS = 16
L = 16
NW = NC * NS
BPW = B // NW
CHUNK = 128
NCHUNK = BPW // CHUNK
NG = CHUNK // L

_mesh = plsc.VectorSubcoreMesh(core_axis_name="c", subcore_axis_name="s")


@functools.partial(
    pl.kernel,
    out_type=jax.ShapeDtypeStruct((B,), jnp.float32),
    mesh=_mesh,
    compiler_params=pltpu.CompilerParams(needs_layout_passes=False),
    scratch_types=[
        pltpu.VMEM((2 * 3 * CHUNK,), jnp.int32),
        pltpu.VMEM((2, CHUNK), jnp.int32),
        pltpu.VMEM((2, CHUNK), jnp.int32),
        pltpu.VMEM((2, CHUNK), jnp.int32),
        pltpu.VMEM((2, CHUNK, D), jnp.float32),
        pltpu.VMEM((2, CHUNK, D), jnp.float32),
        pltpu.VMEM((2, CHUNK, D), jnp.float32),
        pltpu.VMEM((CHUNK,), jnp.float32),
        pltpu.SemaphoreType.DMA,
        pltpu.SemaphoreType.DMA,
    ],
)
def _sc_score(sampf_hbm, ent_hbm, rel_hbm, out_hbm,
              samp_v, idxh_v, idxr_v, idxt_v, hrows_v, rrows_v, trows_v,
              score_v, sem0, sem1):
    wid = lax.axis_index("s") * NC + lax.axis_index("c")
    base = wid * BPW
    iota = lax.iota(jnp.int32, L)
    iota3 = iota * 3
    sems = (sem0, sem1)

    def stage(c, buf):
        cb = base + c * CHUNK
        sbuf = samp_v.at[pl.ds(buf * 3 * CHUNK, 3 * CHUNK)]
        pltpu.sync_copy(sampf_hbm.at[pl.ds(cb * 3, 3 * CHUNK)], sbuf)
        for g in range(NG):
            rows3 = g * (3 * L) + iota3
            idxh_v[buf, pl.ds(g * L, L)] = plsc.load_gather(sbuf, [rows3])
            idxr_v[buf, pl.ds(g * L, L)] = plsc.load_gather(sbuf, [rows3 + 1])
            idxt_v[buf, pl.ds(g * L, L)] = plsc.load_gather(sbuf, [rows3 + 2])
        pltpu.async_copy(ent_hbm.at[idxh_v.at[buf]], hrows_v.at[buf],
                         sems[buf])
        pltpu.async_copy(rel_hbm.at[idxr_v.at[buf]], rrows_v.at[buf],
                         sems[buf])
        pltpu.async_copy(ent_hbm.at[idxt_v.at[buf]], trows_v.at[buf],
                         sems[buf])

    def drain(buf):
        pltpu.make_async_copy(ent_hbm.at[idxh_v.at[buf]], hrows_v.at[buf],
                              sems[buf]).wait()
        pltpu.make_async_copy(rel_hbm.at[idxr_v.at[buf]], rrows_v.at[buf],
                              sems[buf]).wait()
        pltpu.make_async_copy(ent_hbm.at[idxt_v.at[buf]], trows_v.at[buf],
                              sems[buf]).wait()

    def score_chunk(c, buf):
        cb = base + c * CHUNK
        hb, rb, tb = hrows_v.at[buf], rrows_v.at[buf], trows_v.at[buf]

        def gbody(g, _):
            def sbody(j, svec):
                s = g * L + j
                acc = jnp.zeros((L,), jnp.float32)
                for k in range(D // L):
                    sl = pl.ds(k * L, L)
                    acc = acc + jnp.abs(hb[s, sl] + rb[s, sl] - tb[s, sl])
                total = GAMMA - jnp.sum(acc)
                return jnp.where(iota == j, total, svec)

            svec = lax.fori_loop(0, L, sbody, jnp.zeros((L,), jnp.float32),
                                 unroll=2)
            score_v[pl.ds(g * L, L)] = svec
            return 0

        lax.fori_loop(0, NG, gbody, 0)
        pltpu.sync_copy(score_v, out_hbm.at[pl.ds(cb, CHUNK)])

    stage(0, 0)

    def chunk_pair(k, _):
        c = 2 * k
        stage(c + 1, 1)
        drain(0)
        score_chunk(c, 0)

        @pl.when(c + 2 < NCHUNK)
        def _():
            stage(c + 2, 0)

        drain(1)
        score_chunk(c + 1, 1)
        return 0

    lax.fori_loop(0, NCHUNK // 2, chunk_pair, 0)


def kernel(sample, entity_embedding, relation_embedding):
    sample_flat = jnp.reshape(sample.astype(jnp.int32), (3 * B,))
    scores = _sc_score(sample_flat, entity_embedding, relation_embedding)
    return scores[:, None]


# speedup vs baseline: 1.1380x; 1.1380x over previous
import functools

import jax
import jax.numpy as jnp
from jax import lax
from jax.experimental import pallas as pl
from jax.experimental.pallas import tpu as pltpu
from jax.experimental.pallas import tpu_sc as plsc

B = 16384
D = 128
GAMMA = 12.0

NC = 2
NS = 16
L = 16
NW = NC * NS
BPW = B // NW
CHUNK = 128
NCHUNK = BPW // CHUNK
NG = CHUNK // L

_mesh = plsc.VectorSubcoreMesh(core_axis_name="c", subcore_axis_name="s")


@functools.partial(
    pl.kernel,
    out_type=jax.ShapeDtypeStruct((B,), jnp.float32),
    mesh=_mesh,
    compiler_params=pltpu.CompilerParams(needs_layout_passes=False),
    scratch_types=[
        pltpu.VMEM((CHUNK, 3), jnp.int32),
        pltpu.VMEM((2, CHUNK), jnp.int32),
        pltpu.VMEM((2, CHUNK), jnp.int32),
        pltpu.VMEM((2, CHUNK), jnp.int32),
        pltpu.VMEM((2, CHUNK, D), jnp.float32),
        pltpu.VMEM((2, CHUNK, D), jnp.float32),
        pltpu.VMEM((2, CHUNK, D), jnp.float32),
        pltpu.VMEM((CHUNK,), jnp.float32),
        pltpu.SemaphoreType.DMA,
        pltpu.SemaphoreType.DMA,
    ],
)
def _sc_score(samp_hbm, ent_hbm, rel_hbm, out_hbm,
              samp_v, idxh_v, idxr_v, idxt_v, hrows_v, rrows_v, trows_v,
              score_v, sem0, sem1):
    wid = lax.axis_index("s") * NC + lax.axis_index("c")
    base = wid * BPW
    iota = lax.iota(jnp.int32, L)
    sems = (sem0, sem1)
    col0 = jnp.full((L,), 0, jnp.int32)
    col1 = jnp.full((L,), 1, jnp.int32)
    col2 = jnp.full((L,), 2, jnp.int32)

    def stage(c, buf):
        cb = base + c * CHUNK
        pltpu.sync_copy(samp_hbm.at[pl.ds(cb, CHUNK), :], samp_v)
        for g in range(NG):
            rows = g * L + iota
            idxh_v[buf, pl.ds(g * L, L)] = plsc.load_gather(samp_v, [rows, col0])
            idxr_v[buf, pl.ds(g * L, L)] = plsc.load_gather(samp_v, [rows, col1])
            idxt_v[buf, pl.ds(g * L, L)] = plsc.load_gather(samp_v, [rows, col2])
        pltpu.async_copy(ent_hbm.at[idxh_v.at[buf]], hrows_v.at[buf],
                         sems[buf])
        pltpu.async_copy(rel_hbm.at[idxr_v.at[buf]], rrows_v.at[buf],
                         sems[buf])
        pltpu.async_copy(ent_hbm.at[idxt_v.at[buf]], trows_v.at[buf],
                         sems[buf])

    def drain(buf):
        pltpu.make_async_copy(ent_hbm.at[idxh_v.at[buf]], hrows_v.at[buf],
                              sems[buf]).wait()
        pltpu.make_async_copy(rel_hbm.at[idxr_v.at[buf]], rrows_v.at[buf],
                              sems[buf]).wait()
        pltpu.make_async_copy(ent_hbm.at[idxt_v.at[buf]], trows_v.at[buf],
                              sems[buf]).wait()

    def score_chunk(c, buf):
        cb = base + c * CHUNK
        hb, rb, tb = hrows_v.at[buf], rrows_v.at[buf], trows_v.at[buf]

        def gbody(g, _):
            def sbody(j, svec):
                s = g * L + j
                acc = jnp.zeros((L,), jnp.float32)
                for k in range(D // L):
                    sl = pl.ds(k * L, L)
                    acc = acc + jnp.abs(hb[s, sl] + rb[s, sl] - tb[s, sl])
                total = GAMMA - jnp.sum(acc)
                return jnp.where(iota == j, total, svec)

            svec = lax.fori_loop(0, L, sbody, jnp.zeros((L,), jnp.float32),
                                 unroll=2)
            score_v[pl.ds(g * L, L)] = svec
            return 0

        lax.fori_loop(0, NG, gbody, 0)
        pltpu.sync_copy(score_v, out_hbm.at[pl.ds(cb, CHUNK)])

    stage(0, 0)

    def chunk_pair(k, _):
        c = 2 * k
        stage(c + 1, 1)
        drain(0)
        score_chunk(c, 0)

        @pl.when(c + 2 < NCHUNK)
        def _():
            stage(c + 2, 0)

        drain(1)
        score_chunk(c + 1, 1)
        return 0

    lax.fori_loop(0, NCHUNK // 2, chunk_pair, 0)


def kernel(sample, entity_embedding, relation_embedding):
    scores = _sc_score(sample.astype(jnp.int32), entity_embedding,
                       relation_embedding)
    return scores[:, None]
